# baseline (speedup 1.0000x reference)
import functools

import jax
import jax.numpy as jnp
from jax import lax
from jax.experimental import pallas as pl
from jax.experimental.pallas import tpu as pltpu
from jax.experimental.pallas import tpu_sc as plsc

_LANES = 16


def _permute_cols(x, index):
    n_rows, n_cols = x.shape
    info = plsc.get_sparse_core_info()
    num_workers = info.num_cores * info.num_subcores
    rows_per_w = n_rows // num_workers
    chunk = 8
    while rows_per_w % chunk:
        chunk //= 2
    n_chunks = rows_per_w // chunk
    flat_chunk = chunk * n_cols

    mesh = plsc.VectorSubcoreMesh(core_axis_name="c", subcore_axis_name="s")

    @functools.partial(
        pl.kernel,
        out_type=jax.ShapeDtypeStruct((n_rows * n_cols,), jnp.float32),
        mesh=mesh,
        scratch_types=[
            pltpu.VMEM((n_cols,), jnp.int32),
            pltpu.VMEM((flat_chunk,), jnp.float32),
            pltpu.VMEM((flat_chunk,), jnp.float32),
        ],
        compiler_params=pltpu.CompilerParams(needs_layout_passes=False),
    )
    def run(x_hbm, idx_hbm, out_hbm, idx_v, in_v, out_v):
        wid = lax.axis_index("s") * info.num_cores + lax.axis_index("c")
        pltpu.sync_copy(idx_hbm, idx_v)
        base = wid * rows_per_w * n_cols

        def do_chunk(ci, _):
            off0 = pl.multiple_of(base + ci * flat_chunk, 8)
            pltpu.sync_copy(x_hbm.at[pl.ds(off0, flat_chunk)], in_v)

            for r in range(chunk):
                rbase = r * n_cols

                def gather_group(g, _, rbase=rbase):
                    off = pl.multiple_of(g * _LANES, _LANES)
                    cols = idx_v[pl.ds(off, _LANES)]
                    vals = plsc.load_gather(in_v, [cols + rbase])
                    out_v[pl.ds(rbase + off, _LANES)] = vals
                    return 0

                lax.fori_loop(0, n_cols // _LANES, gather_group, 0)

            pltpu.sync_copy(out_v, out_hbm.at[pl.ds(off0, flat_chunk)])
            return 0

        lax.fori_loop(0, n_chunks, do_chunk, 0)

    return run(x.reshape(-1), index).reshape(n_rows, n_cols)


def kernel(x, index):
    z = _permute_cols(x, index)
    log_det = jnp.zeros(x.shape[0], dtype=x.dtype)
    return (z, log_det)


# speedup vs baseline: 2.4074x; 2.4074x over previous
import functools

import jax
import jax.numpy as jnp
from jax import lax
from jax.experimental import pallas as pl
from jax.experimental.pallas import tpu as pltpu
from jax.experimental.pallas import tpu_sc as plsc

_LANES = 16


def _permute_cols(x, index):
    n_rows, n_cols = x.shape
    info = plsc.get_sparse_core_info()
    num_workers = info.num_cores * info.num_subcores
    rows_per_w = n_rows // num_workers
    chunk = 8
    while rows_per_w % chunk:
        chunk //= 2
    n_chunks = rows_per_w // chunk
    flat_chunk = chunk * n_cols

    mesh = plsc.VectorSubcoreMesh(core_axis_name="c", subcore_axis_name="s")

    @functools.partial(
        pl.kernel,
        out_type=jax.ShapeDtypeStruct((n_rows * n_cols,), jnp.float32),
        mesh=mesh,
        scratch_types=[
            pltpu.VMEM((n_cols,), jnp.int32),
            pltpu.VMEM((flat_chunk,), jnp.float32),
            pltpu.VMEM((flat_chunk,), jnp.float32),
        ],
        compiler_params=pltpu.CompilerParams(needs_layout_passes=False),
    )
    def run(x_hbm, idx_hbm, out_hbm, idx_v, in_v, out_v):
        wid = lax.axis_index("s") * info.num_cores + lax.axis_index("c")
        pltpu.sync_copy(idx_hbm, idx_v)
        base = wid * rows_per_w * n_cols

        def do_chunk(ci, _):
            off0 = pl.multiple_of(base + ci * flat_chunk, 8)
            pltpu.sync_copy(x_hbm.at[pl.ds(off0, flat_chunk)], in_v)

            @plsc.parallel_loop(0, n_cols, step=_LANES, unroll=4)
            def gather_group(off):
                off = pl.multiple_of(off, _LANES)
                cols = idx_v[pl.ds(off, _LANES)]
                for r in range(chunk):
                    vals = plsc.load_gather(in_v, [cols + r * n_cols])
                    out_v[pl.ds(off + r * n_cols, _LANES)] = vals

            pltpu.sync_copy(out_v, out_hbm.at[pl.ds(off0, flat_chunk)])
            return 0

        lax.fori_loop(0, n_chunks, do_chunk, 0)

    return run(x.reshape(-1), index).reshape(n_rows, n_cols)


def kernel(x, index):
    z = _permute_cols(x, index)
    log_det = jnp.zeros(x.shape[0], dtype=x.dtype)
    return (z, log_det)


# speedup vs baseline: 2.9187x; 1.2124x over previous
import functools

import jax
import jax.numpy as jnp
from jax import lax
from jax.experimental import pallas as pl
from jax.experimental.pallas import tpu as pltpu
from jax.experimental.pallas import tpu_sc as plsc

_LANES = 16


def _permute_cols(x, index):
    n_rows, n_cols = x.shape
    info = plsc.get_sparse_core_info()
    num_workers = info.num_cores * info.num_subcores
    rows_per_w = n_rows // num_workers
    chunk = 4
    while rows_per_w % (2 * chunk):
        chunk //= 2
    n_chunks = rows_per_w // chunk
    n_pairs = n_chunks // 2
    flat_chunk = chunk * n_cols

    mesh = plsc.VectorSubcoreMesh(core_axis_name="c", subcore_axis_name="s")

    @functools.partial(
        pl.kernel,
        out_type=jax.ShapeDtypeStruct((n_rows * n_cols,), jnp.float32),
        mesh=mesh,
        scratch_types=[
            pltpu.VMEM((n_cols,), jnp.int32),
            [pltpu.VMEM((flat_chunk,), jnp.float32) for _ in range(2)],
            [pltpu.VMEM((flat_chunk,), jnp.float32) for _ in range(2)],
            [pltpu.SemaphoreType.DMA for _ in range(2)],
            [pltpu.SemaphoreType.DMA for _ in range(2)],
        ],
        compiler_params=pltpu.CompilerParams(needs_layout_passes=False),
    )
    def run(x_hbm, idx_hbm, out_hbm, idx_v, in_v, out_v, sem_in, sem_out):
        wid = lax.axis_index("s") * info.num_cores + lax.axis_index("c")
        pltpu.sync_copy(idx_hbm, idx_v)
        base = wid * rows_per_w * n_cols

        def src_at(ci):
            off = pl.multiple_of(base + ci * flat_chunk, 8)
            return x_hbm.at[pl.ds(off, flat_chunk)]

        def dst_at(ci):
            off = pl.multiple_of(base + ci * flat_chunk, 8)
            return out_hbm.at[pl.ds(off, flat_chunk)]

        for b in range(2):
            pltpu.async_copy(src_at(b), in_v[b], sem_in[b])

        def do_pair(pi, _):
            for b in range(2):
                ci = 2 * pi + b
                pltpu.make_async_copy(src_at(ci), in_v[b], sem_in[b]).wait()

                @pl.when(pi > 0)
                def _():
                    pltpu.make_async_copy(
                        out_v[b], dst_at(ci - 2), sem_out[b]
                    ).wait()

                @plsc.parallel_loop(0, n_cols, step=_LANES, unroll=4)
                def gather_group(off):
                    off = pl.multiple_of(off, _LANES)
                    cols = idx_v[pl.ds(off, _LANES)]
                    for r in range(chunk):
                        vals = plsc.load_gather(in_v[b], [cols + r * n_cols])
                        out_v[b][pl.ds(off + r * n_cols, _LANES)] = vals

                pltpu.async_copy(out_v[b], dst_at(ci), sem_out[b])

                @pl.when(pi < n_pairs - 1)
                def _():
                    pltpu.async_copy(src_at(ci + 2), in_v[b], sem_in[b])

            return 0

        lax.fori_loop(0, n_pairs, do_pair, 0)

        for b in range(2):
            ci = n_chunks - 2 + b
            pltpu.make_async_copy(out_v[b], dst_at(ci), sem_out[b]).wait()

    return run(x.reshape(-1), index).reshape(n_rows, n_cols)


def kernel(x, index):
    z = _permute_cols(x, index)
    log_det = jnp.zeros(x.shape[0], dtype=x.dtype)
    return (z, log_det)


# speedup vs baseline: 9.0167x; 3.0893x over previous
import functools

import jax
import jax.numpy as jnp
from jax import lax
from jax.experimental import pallas as pl
from jax.experimental.pallas import tpu as pltpu
from jax.experimental.pallas import tpu_sc as plsc

_LANES = 16


def _permute_cols(x, index):
    n_rows, n_cols = x.shape
    info = plsc.get_sparse_core_info()
    num_workers = info.num_cores * info.num_subcores
    rows_per_w = n_rows // num_workers
    chunk = 4
    while rows_per_w % (2 * chunk):
        chunk //= 2
    n_chunks = rows_per_w // chunk
    n_pairs = n_chunks // 2

    mesh = plsc.VectorSubcoreMesh(core_axis_name="c", subcore_axis_name="s")

    @functools.partial(
        pl.kernel,
        out_type=jax.ShapeDtypeStruct((n_rows, n_cols), jnp.float32),
        mesh=mesh,
        scratch_types=[
            pltpu.VMEM((n_cols,), jnp.int32),
            [pltpu.VMEM((chunk, n_cols), jnp.float32) for _ in range(2)],
            [pltpu.VMEM((chunk, n_cols), jnp.float32) for _ in range(2)],
            [pltpu.SemaphoreType.DMA for _ in range(2)],
            [pltpu.SemaphoreType.DMA for _ in range(2)],
        ],
        compiler_params=pltpu.CompilerParams(needs_layout_passes=False),
    )
    def run(x_hbm, idx_hbm, out_hbm, idx_v, in_v, out_v, sem_in, sem_out):
        wid = lax.axis_index("s") * info.num_cores + lax.axis_index("c")
        pltpu.sync_copy(idx_hbm, idx_v)
        base = wid * rows_per_w

        def src_at(ci):
            return x_hbm.at[pl.ds(base + ci * chunk, chunk)]

        def dst_at(ci):
            return out_hbm.at[pl.ds(base + ci * chunk, chunk)]

        for b in range(2):
            pltpu.async_copy(src_at(b), in_v[b], sem_in[b])

        def do_pair(pi, _):
            for b in range(2):
                ci = 2 * pi + b
                pltpu.make_async_copy(src_at(ci), in_v[b], sem_in[b]).wait()

                @pl.when(pi > 0)
                def _():
                    pltpu.make_async_copy(
                        out_v[b], dst_at(ci - 2), sem_out[b]
                    ).wait()

                @plsc.parallel_loop(0, n_cols, step=_LANES, unroll=4)
                def gather_group(off):
                    off = pl.multiple_of(off, _LANES)
                    cols = idx_v[pl.ds(off, _LANES)]
                    for r in range(chunk):
                        row = jnp.full((_LANES,), r, jnp.int32)
                        vals = plsc.load_gather(in_v[b], [row, cols])
                        out_v[b][r, pl.ds(off, _LANES)] = vals

                pltpu.async_copy(out_v[b], dst_at(ci), sem_out[b])

                @pl.when(pi < n_pairs - 1)
                def _():
                    pltpu.async_copy(src_at(ci + 2), in_v[b], sem_in[b])

            return 0

        lax.fori_loop(0, n_pairs, do_pair, 0)

        for b in range(2):
            ci = n_chunks - 2 + b
            pltpu.make_async_copy(out_v[b], dst_at(ci), sem_out[b]).wait()

    return run(x, index)


def kernel(x, index):
    z = _permute_cols(x, index)
    log_det = jnp.zeros(x.shape[0], dtype=x.dtype)
    return (z, log_det)


# speedup vs baseline: 9.0226x; 1.0007x over previous
import functools

import jax
import jax.numpy as jnp
from jax import lax
from jax.experimental import pallas as pl
from jax.experimental.pallas import tpu as pltpu
from jax.experimental.pallas import tpu_sc as plsc

_LANES = 16


def _permute_cols(x, index):
    n_rows, n_cols = x.shape
    info = plsc.get_sparse_core_info()
    num_workers = info.num_cores * info.num_subcores
    rows_per_w = n_rows // num_workers
    chunk = 4
    while rows_per_w % (2 * chunk):
        chunk //= 2
    n_chunks = rows_per_w // chunk
    n_pairs = n_chunks // 2

    mesh = plsc.VectorSubcoreMesh(core_axis_name="c", subcore_axis_name="s")

    @functools.partial(
        pl.kernel,
        out_type=jax.ShapeDtypeStruct((n_rows, n_cols), jnp.float32),
        mesh=mesh,
        scratch_types=[
            pltpu.VMEM((n_cols,), jnp.int32),
            [pltpu.VMEM((chunk, n_cols), jnp.float32) for _ in range(2)],
            [pltpu.VMEM((chunk, n_cols), jnp.float32) for _ in range(2)],
            [pltpu.SemaphoreType.DMA for _ in range(2)],
            [pltpu.SemaphoreType.DMA for _ in range(2)],
        ],
        compiler_params=pltpu.CompilerParams(needs_layout_passes=False),
    )
    def run(x_hbm, idx_hbm, out_hbm, idx_v, in_v, out_v, sem_in, sem_out):
        wid = lax.axis_index("s") * info.num_cores + lax.axis_index("c")
        pltpu.sync_copy(idx_hbm, idx_v)
        base = wid * rows_per_w

        def src_at(ci):
            return x_hbm.at[pl.ds(base + ci * chunk, chunk)]

        def dst_at(ci):
            return out_hbm.at[pl.ds(base + ci * chunk, chunk)]

        for b in range(2):
            pltpu.async_copy(src_at(b), in_v[b], sem_in[b])

        def do_pair(pi, _):
            for b in range(2):
                ci = 2 * pi + b
                pltpu.make_async_copy(src_at(ci), in_v[b], sem_in[b]).wait()

                @pl.when(pi > 0)
                def _():
                    pltpu.make_async_copy(
                        out_v[b], dst_at(ci - 2), sem_out[b]
                    ).wait()

                @plsc.parallel_loop(0, n_cols, step=_LANES, unroll=8)
                def gather_group(off):
                    off = pl.multiple_of(off, _LANES)
                    cols = idx_v[pl.ds(off, _LANES)]
                    for r in range(chunk):
                        row = jnp.full((_LANES,), r, jnp.int32)
                        vals = plsc.load_gather(in_v[b], [row, cols])
                        out_v[b][r, pl.ds(off, _LANES)] = vals

                pltpu.async_copy(out_v[b], dst_at(ci), sem_out[b])

                @pl.when(pi < n_pairs - 1)
                def _():
                    pltpu.async_copy(src_at(ci + 2), in_v[b], sem_in[b])

            return 0

        lax.fori_loop(0, n_pairs, do_pair, 0)

        for b in range(2):
            ci = n_chunks - 2 + b
            pltpu.make_async_copy(out_v[b], dst_at(ci), sem_out[b]).wait()

    return run(x, index)


def kernel(x, index):
    z = _permute_cols(x, index)
    log_det = jnp.zeros(x.shape[0], dtype=x.dtype)
    return (z, log_det)
